# baseline (speedup 1.0000x reference)
import jax
import jax.numpy as jnp
from jax.experimental import pallas as pl
from jax.experimental.pallas import tpu as pltpu

BLOCK_N = 2560


def _body(h_ref, wt_ref, b_ref, m_ref, o_ref):
    j = pl.program_id(0)
    acc = jax.lax.dot_general(
        h_ref[...], wt_ref[...],
        dimension_numbers=(((1,), (1,)), ((), ())),
        preferred_element_type=jnp.float32,
    )
    bias = b_ref[pl.ds(j * BLOCK_N, BLOCK_N)]
    m = m_ref[pl.ds(j * BLOCK_N, BLOCK_N)]
    o_ref[...] = acc + (bias + (1.0 - m) * -1000000000.0)[None, :]


def kernel(hidden, W, b, mask):
    B, H = hidden.shape
    V = W.shape[1]
    wt = W.T
    nb = pl.cdiv(V, BLOCK_N)
    return pl.pallas_call(
        _body,
        grid=(nb,),
        in_specs=[
            pl.BlockSpec((B, H), lambda j: (0, 0)),
            pl.BlockSpec((BLOCK_N, H), lambda j: (j, 0)),
            pl.BlockSpec((V,), lambda j: (0,)),
            pl.BlockSpec((V,), lambda j: (0,)),
        ],
        out_specs=pl.BlockSpec((B, BLOCK_N), lambda j: (0, j)),
        out_shape=jax.ShapeDtypeStruct((B, V), jnp.float32),
        compiler_params=pltpu.CompilerParams(
            dimension_semantics=("parallel",),
        ),
    )(hidden, wt, b, mask)


# speedup vs baseline: 1.0058x; 1.0058x over previous
import jax
import jax.numpy as jnp
from jax.experimental import pallas as pl
from jax.experimental.pallas import tpu as pltpu

BLOCK_N = 3328


def _body(h_ref, wt_ref, b_ref, m_ref, o_ref):
    j = pl.program_id(0)
    acc = jax.lax.dot_general(
        h_ref[...], wt_ref[...],
        dimension_numbers=(((1,), (1,)), ((), ())),
        preferred_element_type=jnp.float32,
    )
    bias = b_ref[pl.ds(j * BLOCK_N, BLOCK_N)]
    m = m_ref[pl.ds(j * BLOCK_N, BLOCK_N)]
    o_ref[...] = acc + (bias + (1.0 - m) * -1000000000.0)[None, :]


def kernel(hidden, W, b, mask):
    B, H = hidden.shape
    V = W.shape[1]
    wt = W.T
    nb = pl.cdiv(V, BLOCK_N)
    return pl.pallas_call(
        _body,
        grid=(nb,),
        in_specs=[
            pl.BlockSpec((B, H), lambda j: (0, 0)),
            pl.BlockSpec((BLOCK_N, H), lambda j: (j, 0)),
            pl.BlockSpec((V,), lambda j: (0,)),
            pl.BlockSpec((V,), lambda j: (0,)),
        ],
        out_specs=pl.BlockSpec((B, BLOCK_N), lambda j: (0, j)),
        out_shape=jax.ShapeDtypeStruct((B, V), jnp.float32),
        compiler_params=pltpu.CompilerParams(
            dimension_semantics=("parallel",),
        ),
    )(hidden, wt, b, mask)


# speedup vs baseline: 1.0169x; 1.0110x over previous
import jax
import jax.numpy as jnp
from jax.experimental import pallas as pl
from jax.experimental.pallas import tpu as pltpu

BLOCK_N = 3200


def _body(h_ref, wt_ref, b_ref, m_ref, o_ref):
    j = pl.program_id(0)
    acc = jax.lax.dot_general(
        h_ref[...], wt_ref[...],
        dimension_numbers=(((1,), (1,)), ((), ())),
        preferred_element_type=jnp.float32,
    )
    bias = b_ref[pl.ds(j * BLOCK_N, BLOCK_N)]
    m = m_ref[pl.ds(j * BLOCK_N, BLOCK_N)]
    o_ref[...] = acc + (bias + (1.0 - m) * -1000000000.0)[None, :]


def kernel(hidden, W, b, mask):
    B, H = hidden.shape
    V = W.shape[1]
    wt = W.T
    nb = pl.cdiv(V, BLOCK_N)
    return pl.pallas_call(
        _body,
        grid=(nb,),
        in_specs=[
            pl.BlockSpec((B, H), lambda j: (0, 0)),
            pl.BlockSpec((BLOCK_N, H), lambda j: (j, 0)),
            pl.BlockSpec((V,), lambda j: (0,)),
            pl.BlockSpec((V,), lambda j: (0,)),
        ],
        out_specs=pl.BlockSpec((B, BLOCK_N), lambda j: (0, j)),
        out_shape=jax.ShapeDtypeStruct((B, V), jnp.float32),
        compiler_params=pltpu.CompilerParams(
            dimension_semantics=("parallel",),
        ),
    )(hidden, wt, b, mask)


# speedup vs baseline: 1.0235x; 1.0065x over previous
import jax
import jax.numpy as jnp
from jax.experimental import pallas as pl
from jax.experimental.pallas import tpu as pltpu

BLOCK_N = 3072


def _body(h_ref, wt_ref, b_ref, m_ref, o_ref):
    j = pl.program_id(0)
    acc = jax.lax.dot_general(
        h_ref[...], wt_ref[...],
        dimension_numbers=(((1,), (1,)), ((), ())),
        preferred_element_type=jnp.float32,
    )
    bias = b_ref[pl.ds(j * BLOCK_N, BLOCK_N)]
    m = m_ref[pl.ds(j * BLOCK_N, BLOCK_N)]
    o_ref[...] = acc + (bias + (1.0 - m) * -1000000000.0)[None, :]


def kernel(hidden, W, b, mask):
    B, H = hidden.shape
    V = W.shape[1]
    wt = W.T
    nb = pl.cdiv(V, BLOCK_N)
    return pl.pallas_call(
        _body,
        grid=(nb,),
        in_specs=[
            pl.BlockSpec((B, H), lambda j: (0, 0)),
            pl.BlockSpec((BLOCK_N, H), lambda j: (j, 0)),
            pl.BlockSpec((V,), lambda j: (0,)),
            pl.BlockSpec((V,), lambda j: (0,)),
        ],
        out_specs=pl.BlockSpec((B, BLOCK_N), lambda j: (0, j)),
        out_shape=jax.ShapeDtypeStruct((B, V), jnp.float32),
        compiler_params=pltpu.CompilerParams(
            dimension_semantics=("arbitrary",),
        ),
    )(hidden, wt, b, mask)


# speedup vs baseline: 1.0286x; 1.0050x over previous
import jax
import jax.numpy as jnp
from jax.experimental import pallas as pl
from jax.experimental.pallas import tpu as pltpu

BLOCK_N = 2944


def _body(h_ref, wt_ref, b_ref, m_ref, o_ref):
    j = pl.program_id(0)
    acc = jax.lax.dot_general(
        h_ref[...], wt_ref[...],
        dimension_numbers=(((1,), (1,)), ((), ())),
        preferred_element_type=jnp.float32,
    )
    bias = b_ref[pl.ds(j * BLOCK_N, BLOCK_N)]
    m = m_ref[pl.ds(j * BLOCK_N, BLOCK_N)]
    o_ref[...] = acc + (bias + (1.0 - m) * -1000000000.0)[None, :]


def kernel(hidden, W, b, mask):
    B, H = hidden.shape
    V = W.shape[1]
    wt = W.T
    nb = pl.cdiv(V, BLOCK_N)
    return pl.pallas_call(
        _body,
        grid=(nb,),
        in_specs=[
            pl.BlockSpec((B, H), lambda j: (0, 0)),
            pl.BlockSpec((BLOCK_N, H), lambda j: (j, 0)),
            pl.BlockSpec((V,), lambda j: (0,)),
            pl.BlockSpec((V,), lambda j: (0,)),
        ],
        out_specs=pl.BlockSpec((B, BLOCK_N), lambda j: (0, j)),
        out_shape=jax.ShapeDtypeStruct((B, V), jnp.float32),
        compiler_params=pltpu.CompilerParams(
            dimension_semantics=("parallel",),
        ),
    )(hidden, wt, b, mask)
